# baseline (speedup 1.0000x reference)
import functools

import jax
import jax.numpy as jnp
from jax import lax
from jax.experimental import pallas as pl
from jax.experimental.pallas import tpu as pltpu
from jax.experimental.pallas import tpu_sc as plsc

DIN = 128
DOUT = 128

NUM_CORES = 2
NUM_TILES = 16
CHUNK = 128

ROW_BLOCK = 1000


def _tc_body(x_ref, w_ref, b_ref, o_ref):
    wx = jnp.dot(x_ref[...], w_ref[...].T, preferred_element_type=jnp.float32)
    o_ref[...] = wx + b_ref[...]


def _project(x, w, b):
    n = x.shape[0]
    return pl.pallas_call(
        _tc_body,
        grid=(n // ROW_BLOCK,),
        in_specs=[
            pl.BlockSpec((ROW_BLOCK, DIN), lambda i: (i, 0)),
            pl.BlockSpec((DOUT, DIN), lambda i: (0, 0)),
            pl.BlockSpec((1, DOUT), lambda i: (0, 0)),
        ],
        out_specs=pl.BlockSpec((ROW_BLOCK, DOUT), lambda i: (i, 0)),
        out_shape=jax.ShapeDtypeStruct((n, DOUT), jnp.float32),
    )(x, w, b.reshape(1, DOUT))


def _combine_body(p_ref, o_ref):
    o_ref[...] = p_ref[0] + p_ref[1]


def _combine(partials, n):
    return pl.pallas_call(
        _combine_body,
        grid=(n // ROW_BLOCK,),
        in_specs=[pl.BlockSpec((NUM_CORES, ROW_BLOCK, DOUT),
                               lambda i: (0, i, 0))],
        out_specs=pl.BlockSpec((ROW_BLOCK, DOUT), lambda i: (i, 0)),
        out_shape=jax.ShapeDtypeStruct((n, DOUT), jnp.float32),
    )(partials)


def _make_sc_spmm(n_pad, n_chunks):
    rows_per_tile = n_pad // NUM_TILES
    mesh = plsc.VectorSubcoreMesh(
        core_axis_name="c", subcore_axis_name="s",
        num_cores=NUM_CORES, num_subcores=NUM_TILES)

    @functools.partial(
        pl.kernel,
        out_type=jax.ShapeDtypeStruct((NUM_CORES, n_pad, DOUT), jnp.float32),
        mesh=mesh,
        scratch_types=[
            pltpu.VMEM((n_chunks, CHUNK), jnp.int32),
            pltpu.VMEM((n_chunks, CHUNK), jnp.int32),
            pltpu.VMEM((n_chunks, CHUNK), jnp.float32),
            pltpu.VMEM((CHUNK, DOUT), jnp.float32),
            pltpu.VMEM_SHARED((n_pad, DOUT), jnp.float32),
            pltpu.SemaphoreType.DMA,
        ],
    )
    def sc_spmm(wx, cols_h, rows_h, vals_h, out,
                cols_t, rows_t, vals_t, gbuf, acc, sem):
        cid = lax.axis_index("c")
        sid = lax.axis_index("s")
        wid = cid * NUM_TILES + sid

        pltpu.sync_copy(cols_h.at[wid], cols_t)
        pltpu.sync_copy(rows_h.at[wid], rows_t)
        pltpu.sync_copy(vals_h.at[wid], vals_t)

        zero = jnp.zeros((16,), jnp.float32)
        per_row = DOUT // 16

        def zero_gbuf(i, _):
            gbuf[lax.div(i, per_row), pl.ds(lax.rem(i, per_row) * 16, 16)] = zero
            return 0

        lax.fori_loop(0, CHUNK * per_row, zero_gbuf, 0)

        base = sid * rows_per_tile

        def zero_acc(k, _):
            pltpu.sync_copy(gbuf, acc.at[pl.ds(base + k * CHUNK, CHUNK)])
            return 0

        lax.fori_loop(0, rows_per_tile // CHUNK, zero_acc, 0)

        plsc.subcore_barrier()

        def chunk_body(j, _):
            pltpu.async_copy(wx.at[cols_t.at[j]], gbuf, sem).wait()

            def scale(g, _):
                v16 = vals_t[j, pl.ds(g * 16, 16)]
                for l in range(16):
                    vb = jnp.full((16,), v16[l], jnp.float32)
                    e = g * 16 + l
                    for q in range(per_row):
                        sl = pl.ds(q * 16, 16)
                        gbuf[e, sl] = gbuf[e, sl] * vb
                return 0

            lax.fori_loop(0, CHUNK // 16, scale, 0)

            pltpu.sync_copy(gbuf, acc.at[rows_t.at[j]], add=True)
            return 0

        lax.fori_loop(0, n_chunks, chunk_body, 0)

        plsc.subcore_barrier()
        pltpu.sync_copy(acc.at[pl.ds(base, rows_per_tile)],
                        out.at[cid, pl.ds(base, rows_per_tile)])

    return sc_spmm


def kernel(A_indices, A_values, X, W, b):
    e = A_values.shape[0]
    n = X.shape[0]
    n_workers = NUM_CORES * NUM_TILES

    wx = _project(X, W, b)

    per_tile = -(-e // (n_workers * CHUNK)) * CHUNK
    n_chunks = per_tile // CHUNK
    pad = n_workers * per_tile - e

    rows = A_indices[0]
    cols = A_indices[1]
    if pad:
        zpad = jnp.zeros((pad,), jnp.int32)
        rows = jnp.concatenate([rows, zpad])
        cols = jnp.concatenate([cols, zpad])
        vals = jnp.concatenate([A_values, jnp.zeros((pad,), jnp.float32)])
    else:
        vals = A_values
    cols_h = cols.reshape(n_workers, n_chunks, CHUNK)
    rows_h = rows.reshape(n_workers, n_chunks, CHUNK)
    vals_h = vals.reshape(n_workers, n_chunks, CHUNK)

    n_pad = -(-n // (NUM_TILES * CHUNK)) * (NUM_TILES * CHUNK)
    partials = _make_sc_spmm(n_pad, n_chunks)(wx, cols_h, rows_h, vals_h)
    return _combine(partials, n)


# speedup vs baseline: 1.1648x; 1.1648x over previous
import functools

import jax
import jax.numpy as jnp
from jax import lax
from jax.experimental import pallas as pl
from jax.experimental.pallas import tpu as pltpu
from jax.experimental.pallas import tpu_sc as plsc

DIN = 128
DOUT = 128

NUM_CORES = 2
NUM_TILES = 16
CHUNK = 128

ROW_BLOCK = 1000


def _tc_body(x_ref, w_ref, b_ref, o_ref):
    wx = jnp.dot(x_ref[...], w_ref[...].T, preferred_element_type=jnp.float32)
    o_ref[...] = wx + b_ref[...]


def _project(x, w, b):
    n = x.shape[0]
    return pl.pallas_call(
        _tc_body,
        grid=(n // ROW_BLOCK,),
        in_specs=[
            pl.BlockSpec((ROW_BLOCK, DIN), lambda i: (i, 0)),
            pl.BlockSpec((DOUT, DIN), lambda i: (0, 0)),
            pl.BlockSpec((1, DOUT), lambda i: (0, 0)),
        ],
        out_specs=pl.BlockSpec((ROW_BLOCK, DOUT), lambda i: (i, 0)),
        out_shape=jax.ShapeDtypeStruct((n, DOUT), jnp.float32),
    )(x, w, b.reshape(1, DOUT))


def _combine_body(p_ref, o_ref):
    o_ref[...] = p_ref[0] + p_ref[1]


def _combine(partials, n):
    return pl.pallas_call(
        _combine_body,
        grid=(n // ROW_BLOCK,),
        in_specs=[pl.BlockSpec((NUM_CORES, ROW_BLOCK, DOUT),
                               lambda i: (0, i, 0))],
        out_specs=pl.BlockSpec((ROW_BLOCK, DOUT), lambda i: (i, 0)),
        out_shape=jax.ShapeDtypeStruct((n, DOUT), jnp.float32),
    )(partials)


def _make_sc_spmm(n_pad, n_chunks):
    rows_per_tile = n_pad // NUM_TILES
    mesh = plsc.VectorSubcoreMesh(
        core_axis_name="c", subcore_axis_name="s",
        num_cores=NUM_CORES, num_subcores=NUM_TILES)

    @functools.partial(
        pl.kernel,
        out_type=jax.ShapeDtypeStruct((NUM_CORES, n_pad, DOUT), jnp.float32),
        mesh=mesh,
        scratch_types=[
            pltpu.VMEM((n_chunks, CHUNK), jnp.int32),
            pltpu.VMEM((n_chunks, CHUNK), jnp.int32),
            pltpu.VMEM((n_chunks, CHUNK), jnp.float32),
            pltpu.VMEM((CHUNK, DOUT), jnp.float32),
            pltpu.VMEM_SHARED((n_pad, DOUT), jnp.float32),
            pltpu.SemaphoreType.DMA,
        ],
    )
    def sc_spmm(wx, cols_h, rows_h, vals_h, out,
                cols_t, rows_t, vals_t, gbuf, acc, sem):
        cid = lax.axis_index("c")
        sid = lax.axis_index("s")
        wid = cid * NUM_TILES + sid

        pltpu.sync_copy(cols_h.at[wid], cols_t)
        pltpu.sync_copy(rows_h.at[wid], rows_t)
        pltpu.sync_copy(vals_h.at[wid], vals_t)

        zero = jnp.zeros((16,), jnp.float32)
        per_row = DOUT // 16

        def zero_gbuf(i, _):
            gbuf[lax.div(i, per_row), pl.ds(lax.rem(i, per_row) * 16, 16)] = zero
            return 0

        lax.fori_loop(0, CHUNK * per_row, zero_gbuf, 0)

        base = sid * rows_per_tile

        def zero_acc(k, _):
            pltpu.sync_copy(gbuf, acc.at[pl.ds(base + k * CHUNK, CHUNK)])
            return 0

        lax.fori_loop(0, rows_per_tile // CHUNK, zero_acc, 0)

        plsc.subcore_barrier()

        def chunk_body(j, _):
            pltpu.async_copy(wx.at[cols_t.at[j]], gbuf, sem).wait()

            @functools.partial(plsc.parallel_loop, 0, CHUNK // 16,
                               unroll=2)
            def _(g):
                v16 = vals_t[j, pl.ds(g * 16, 16)]
                for l in range(16):
                    vb = jnp.full((16,), v16[l], jnp.float32)
                    e = g * 16 + l
                    for q in range(per_row):
                        sl = pl.ds(q * 16, 16)
                        gbuf[e, sl] = gbuf[e, sl] * vb

            pltpu.sync_copy(gbuf, acc.at[rows_t.at[j]], add=True)
            return 0

        lax.fori_loop(0, n_chunks, chunk_body, 0)

        plsc.subcore_barrier()
        pltpu.sync_copy(acc.at[pl.ds(base, rows_per_tile)],
                        out.at[cid, pl.ds(base, rows_per_tile)])

    return sc_spmm


def kernel(A_indices, A_values, X, W, b):
    e = A_values.shape[0]
    n = X.shape[0]
    n_workers = NUM_CORES * NUM_TILES

    wx = _project(X, W, b)

    per_tile = -(-e // (n_workers * CHUNK)) * CHUNK
    n_chunks = per_tile // CHUNK
    pad = n_workers * per_tile - e

    rows = A_indices[0]
    cols = A_indices[1]
    if pad:
        zpad = jnp.zeros((pad,), jnp.int32)
        rows = jnp.concatenate([rows, zpad])
        cols = jnp.concatenate([cols, zpad])
        vals = jnp.concatenate([A_values, jnp.zeros((pad,), jnp.float32)])
    else:
        vals = A_values
    cols_h = cols.reshape(n_workers, n_chunks, CHUNK)
    rows_h = rows.reshape(n_workers, n_chunks, CHUNK)
    vals_h = vals.reshape(n_workers, n_chunks, CHUNK)

    n_pad = -(-n // (NUM_TILES * CHUNK)) * (NUM_TILES * CHUNK)
    partials = _make_sc_spmm(n_pad, n_chunks)(wx, cols_h, rows_h, vals_h)
    return _combine(partials, n)
